# baseline (speedup 1.0000x reference)
import functools

import jax
import jax.numpy as jnp
from jax import lax
from jax.experimental import pallas as pl
from jax.experimental.pallas import tpu as pltpu
from jax.experimental.pallas import tpu_sc as plsc

_N_WORKERS = 32
_CHUNK = 200


def _sc_edge_sum(n_rows, d_hid):
    rpw = n_rows // _N_WORKERS
    nch = rpw // _CHUNK
    n_lane_blocks = d_hid // 16
    mesh = plsc.VectorSubcoreMesh(core_axis_name="c", subcore_axis_name="s")

    @functools.partial(
        pl.kernel,
        mesh=mesh,
        out_type=jax.ShapeDtypeStruct((_N_WORKERS, d_hid), jnp.float32),
        scratch_types=[
            pltpu.VMEM((_CHUNK, d_hid), jnp.float32),
            pltpu.VMEM((_CHUNK, d_hid), jnp.float32),
            pltpu.VMEM((d_hid,), jnp.float32),
            pltpu.SemaphoreType.DMA,
            pltpu.SemaphoreType.DMA,
        ],
    )
    def body(e_hbm, out_hbm, buf0, buf1, acc, sem0, sem1):
        wid = lax.axis_index("s") * 2 + lax.axis_index("c")
        base = wid * rpw
        for d in range(n_lane_blocks):
            acc[pl.ds(d * 16, 16)] = jnp.zeros((16,), jnp.float32)
        bufs = (buf0, buf1)
        sems = (sem0, sem1)
        pltpu.async_copy(e_hbm.at[pl.ds(base, _CHUNK)], buf0, sem0)

        def accumulate(cur):
            def row(r, carry):
                for d in range(n_lane_blocks):
                    sl = pl.ds(d * 16, 16)
                    plsc.addupdate(acc.at[sl], cur[r, sl])
                return carry
            lax.fori_loop(0, _CHUNK, row, 0)

        def pair(k2, carry):
            for b in range(2):
                k = k2 * 2 + b
                cur, csem = bufs[b], sems[b]
                nxt, nsem = bufs[1 - b], sems[1 - b]

                @pl.when(k + 1 < nch)
                def _start_next():
                    pltpu.async_copy(
                        e_hbm.at[pl.ds(base + (k + 1) * _CHUNK, _CHUNK)],
                        nxt, nsem)

                pltpu.make_async_copy(
                    e_hbm.at[pl.ds(base, _CHUNK)], cur, csem).wait()
                accumulate(cur)
            return carry

        lax.fori_loop(0, nch // 2, pair, 0)
        pltpu.sync_copy(acc, out_hbm.at[wid])

    return body


def _tc_body(x_ref, e_ref, out_ref, accx_ref, acce_ref, *, grid):
    i = pl.program_id(0)

    @pl.when(i == 0)
    def _init():
        accx_ref[...] = jnp.zeros_like(accx_ref)
        acce_ref[...] = jnp.zeros_like(acce_ref)

    accx_ref[...] += jnp.sum(x_ref[...], axis=0, keepdims=True)
    acce_ref[...] += jnp.sum(e_ref[...], axis=0, keepdims=True)

    @pl.when(i == grid - 1)
    def _finish():
        out_ref[0:1, :] = accx_ref[...]
        out_ref[1:2, 0:acce_ref.shape[1]] = acce_ref[...]


def _combine_body(sums_ref, scp_ref, w_ref, out_ref, *, d_in, d_hid):
    sx = sums_ref[0:1, :]
    se = sums_ref[1:2, 0:d_hid]
    se += jnp.sum(scp_ref[...], axis=0, keepdims=True)
    r = jnp.dot(sx, w_ref[:d_in, :], preferred_element_type=jnp.float32)
    r += jnp.dot(se, w_ref[d_in:, :], preferred_element_type=jnp.float32)
    out_ref[...] = r


def kernel(x, edge_hidden, edge_index, W):
    del edge_index
    n_nodes, d_in = x.shape
    n_edges, d_hid = edge_hidden.shape

    sc_rows = 64000
    tc_rows = n_edges - sc_rows
    grid = 25
    bx = n_nodes // grid
    be = tc_rows // grid

    sc_partials = _sc_edge_sum(sc_rows, d_hid)(edge_hidden[:sc_rows])

    tc_body = functools.partial(_tc_body, grid=grid)
    sums = pl.pallas_call(
        tc_body,
        grid=(grid,),
        in_specs=[
            pl.BlockSpec((bx, d_in), lambda i: (i, 0)),
            pl.BlockSpec((be, d_hid), lambda i: (i, 0)),
        ],
        out_specs=pl.BlockSpec((2, d_in), lambda i: (0, 0)),
        out_shape=jax.ShapeDtypeStruct((2, d_in), jnp.float32),
        scratch_shapes=[
            pltpu.VMEM((1, d_in), jnp.float32),
            pltpu.VMEM((1, d_hid), jnp.float32),
        ],
    )(x, edge_hidden[sc_rows:])

    combine = functools.partial(_combine_body, d_in=d_in, d_hid=d_hid)
    out = pl.pallas_call(
        combine,
        out_shape=jax.ShapeDtypeStruct((1, d_hid), jnp.float32),
    )(sums, sc_partials, W)
    return out[0]


# speedup vs baseline: 1.0224x; 1.0224x over previous
import functools

import jax
import jax.numpy as jnp
from jax import lax
from jax.experimental import pallas as pl
from jax.experimental.pallas import tpu as pltpu
from jax.experimental.pallas import tpu_sc as plsc

_N_WORKERS = 32
_CHUNK = 200


def _sc_edge_sum(n_rows, d_hid):
    rpw = n_rows // _N_WORKERS
    nch = rpw // _CHUNK
    n_lane_blocks = d_hid // 16
    mesh = plsc.VectorSubcoreMesh(core_axis_name="c", subcore_axis_name="s")

    @functools.partial(
        pl.kernel,
        mesh=mesh,
        out_type=jax.ShapeDtypeStruct((_N_WORKERS, d_hid), jnp.float32),
        scratch_types=[
            pltpu.VMEM((_CHUNK, d_hid), jnp.float32),
            pltpu.VMEM((_CHUNK, d_hid), jnp.float32),
            pltpu.VMEM((d_hid,), jnp.float32),
            pltpu.SemaphoreType.DMA,
            pltpu.SemaphoreType.DMA,
        ],
    )
    def body(e_hbm, out_hbm, buf0, buf1, acc, sem0, sem1):
        wid = lax.axis_index("s") * 2 + lax.axis_index("c")
        base = wid * rpw
        for d in range(n_lane_blocks):
            acc[pl.ds(d * 16, 16)] = jnp.zeros((16,), jnp.float32)
        bufs = (buf0, buf1)
        sems = (sem0, sem1)
        pltpu.async_copy(e_hbm.at[pl.ds(base, _CHUNK)], buf0, sem0)

        def accumulate(cur):
            def grp(g, carry):
                r0 = g * 8
                for d in range(n_lane_blocks):
                    sl = pl.ds(d * 16, 16)
                    v = cur[r0, sl]
                    for rr in range(1, 8):
                        v = v + cur[r0 + rr, sl]
                    plsc.addupdate(acc.at[sl], v)
                return carry
            lax.fori_loop(0, _CHUNK // 8, grp, 0)

        def pair(k2, carry):
            for b in range(2):
                k = k2 * 2 + b
                cur, csem = bufs[b], sems[b]
                nxt, nsem = bufs[1 - b], sems[1 - b]

                @pl.when(k + 1 < nch)
                def _start_next():
                    pltpu.async_copy(
                        e_hbm.at[pl.ds(base + (k + 1) * _CHUNK, _CHUNK)],
                        nxt, nsem)

                pltpu.make_async_copy(
                    e_hbm.at[pl.ds(base, _CHUNK)], cur, csem).wait()
                accumulate(cur)
            return carry

        lax.fori_loop(0, nch // 2, pair, 0)
        pltpu.sync_copy(acc, out_hbm.at[wid])

    return body


def _tc_body(x_ref, e_ref, out_ref, accx_ref, acce_ref, *, grid):
    i = pl.program_id(0)

    @pl.when(i == 0)
    def _init():
        accx_ref[...] = jnp.zeros_like(accx_ref)
        acce_ref[...] = jnp.zeros_like(acce_ref)

    accx_ref[...] += jnp.sum(x_ref[...], axis=0, keepdims=True)
    acce_ref[...] += jnp.sum(e_ref[...], axis=0, keepdims=True)

    @pl.when(i == grid - 1)
    def _finish():
        out_ref[0:1, :] = accx_ref[...]
        out_ref[1:2, 0:acce_ref.shape[1]] = acce_ref[...]


def _combine_body(sums_ref, scp_ref, w_ref, out_ref, *, d_in, d_hid):
    sx = sums_ref[0:1, :]
    se = sums_ref[1:2, 0:d_hid]
    se += jnp.sum(scp_ref[...], axis=0, keepdims=True)
    r = jnp.dot(sx, w_ref[:d_in, :], preferred_element_type=jnp.float32)
    r += jnp.dot(se, w_ref[d_in:, :], preferred_element_type=jnp.float32)
    out_ref[...] = r


def kernel(x, edge_hidden, edge_index, W):
    del edge_index
    n_nodes, d_in = x.shape
    n_edges, d_hid = edge_hidden.shape

    sc_rows = 64000
    tc_rows = n_edges - sc_rows
    grid = 25
    bx = n_nodes // grid
    be = tc_rows // grid

    sc_partials = _sc_edge_sum(sc_rows, d_hid)(edge_hidden[:sc_rows])

    tc_body = functools.partial(_tc_body, grid=grid)
    sums = pl.pallas_call(
        tc_body,
        grid=(grid,),
        in_specs=[
            pl.BlockSpec((bx, d_in), lambda i: (i, 0)),
            pl.BlockSpec((be, d_hid), lambda i: (i, 0)),
        ],
        out_specs=pl.BlockSpec((2, d_in), lambda i: (0, 0)),
        out_shape=jax.ShapeDtypeStruct((2, d_in), jnp.float32),
        scratch_shapes=[
            pltpu.VMEM((1, d_in), jnp.float32),
            pltpu.VMEM((1, d_hid), jnp.float32),
        ],
    )(x, edge_hidden[sc_rows:])

    combine = functools.partial(_combine_body, d_in=d_in, d_hid=d_hid)
    out = pl.pallas_call(
        combine,
        out_shape=jax.ShapeDtypeStruct((1, d_hid), jnp.float32),
    )(sums, sc_partials, W)
    return out[0]


# speedup vs baseline: 3.2894x; 3.2173x over previous
import functools

import jax
import jax.numpy as jnp
from jax.experimental import pallas as pl
from jax.experimental.pallas import tpu as pltpu


def _body(x_ref, e_ref, w_ref, out_ref, accx_ref, acce_ref, *, grid, d_in):
    i = pl.program_id(0)

    @pl.when(i == 0)
    def _init():
        accx_ref[...] = jnp.zeros_like(accx_ref)
        acce_ref[...] = jnp.zeros_like(acce_ref)

    accx_ref[...] += jnp.sum(x_ref[...], axis=0, keepdims=True)
    acce_ref[...] += jnp.sum(e_ref[...], axis=0, keepdims=True)

    @pl.when(i == grid - 1)
    def _finish():
        sx = accx_ref[...]
        se = acce_ref[...]
        r = jnp.dot(sx, w_ref[:d_in, :], preferred_element_type=jnp.float32)
        r += jnp.dot(se, w_ref[d_in:, :], preferred_element_type=jnp.float32)
        out_ref[...] = r


def kernel(x, edge_hidden, edge_index, W):
    del edge_index
    n_nodes, d_in = x.shape
    n_edges, d_hid = edge_hidden.shape
    grid = 25
    bx = n_nodes // grid
    be = n_edges // grid

    body = functools.partial(_body, grid=grid, d_in=d_in)
    out = pl.pallas_call(
        body,
        grid=(grid,),
        in_specs=[
            pl.BlockSpec((bx, d_in), lambda i: (i, 0)),
            pl.BlockSpec((be, d_hid), lambda i: (i, 0)),
            pl.BlockSpec((d_in + d_hid, d_hid), lambda i: (0, 0)),
        ],
        out_specs=pl.BlockSpec((1, d_hid), lambda i: (0, 0)),
        out_shape=jax.ShapeDtypeStruct((1, d_hid), jnp.float32),
        scratch_shapes=[
            pltpu.VMEM((1, d_in), jnp.float32),
            pltpu.VMEM((1, d_hid), jnp.float32),
        ],
    )(x, edge_hidden, W)
    return out[0]
